# Initial kernel scaffold; baseline (speedup 1.0000x reference)
import functools

import jax
import jax.numpy as jnp
from jax import lax
from jax.experimental import pallas as pl
from jax.experimental.pallas import tpu as pltpu
from jax.experimental.pallas import tpu_sc as plsc

_B, _N, _K, _D = 8, 4096, 16, 64
_ROWS = _B * _N * _K
_RPB = _N * _K
_NC, _NS = 2, 16
_NW = _NC * _NS
_RPW = _ROWS // _NW
_C = 1024
_NCHUNK = _RPW // _C
_G = 128
_NG = _C // _G

_mesh = plsc.VectorSubcoreMesh(core_axis_name="c", subcore_axis_name="s")


@functools.partial(
    pl.kernel,
    mesh=_mesh,
    out_type=jax.ShapeDtypeStruct((_ROWS, _D), jnp.float32),
    scratch_types=[
        pltpu.VMEM((_C,), jnp.int32),
        pltpu.VMEM((_C, _D), jnp.float32),
        pltpu.SemaphoreType.DMA,
    ],
)
def _lfi_gather(x_hbm, idx_hbm, out_hbm, idx_v, rows_v, sem):
    cid = lax.axis_index("c")
    sid = lax.axis_index("s")
    wid = sid * _NC + cid
    base = wid * _RPW
    off = (base // _RPB) * _N

    def chunk_body(g, carry):
        row0 = base + g * _C
        pltpu.sync_copy(idx_hbm.at[pl.ds(row0, _C)], idx_v)

        def add_body(i, c):
            s = pl.ds(i * 16, 16)
            idx_v[s] = idx_v[s] + off
            return c

        lax.fori_loop(0, _C // 16, add_body, 0)

        copies = []
        for j in range(_NG):
            copies.append(
                pltpu.async_copy(
                    x_hbm.at[idx_v.at[pl.ds(j * _G, _G)]],
                    rows_v.at[pl.ds(j * _G, _G)],
                    sem,
                )
            )
        for cp in copies:
            cp.wait()
        pltpu.sync_copy(rows_v, out_hbm.at[pl.ds(row0, _C)])
        return carry

    lax.fori_loop(0, _NCHUNK, chunk_body, 0)


def kernel(x, refer_idx):
    b, n, d = x.shape
    k = refer_idx.shape[2]
    x_flat = x.reshape(b * n, d)
    idx_flat = refer_idx.astype(jnp.int32).reshape(-1)
    out = _lfi_gather(x_flat, idx_flat)
    return out.reshape(b, n, k * d)


# speedup vs baseline: 6.5257x; 6.5257x over previous
import functools

import jax
import jax.numpy as jnp
from jax import lax
from jax.experimental import pallas as pl
from jax.experimental.pallas import tpu as pltpu
from jax.experimental.pallas import tpu_sc as plsc

_B, _N, _K, _D = 8, 4096, 16, 64
_ROWS = _B * _N * _K
_RPB = _N * _K
_NC, _NS = 2, 16
_NW = _NC * _NS
_RPW = _ROWS // _NW
_C = 1024
_NCHUNK = _RPW // _C
_G = 128
_NG = _C // _G

_mesh = plsc.VectorSubcoreMesh(core_axis_name="c", subcore_axis_name="s")


@functools.partial(
    pl.kernel,
    mesh=_mesh,
    out_type=jax.ShapeDtypeStruct((_ROWS, _D), jnp.float32),
    scratch_types=[
        pltpu.VMEM((_C,), jnp.int32),
        pltpu.VMEM((_C, _D), jnp.float32),
        pltpu.SemaphoreType.DMA,
    ],
    compiler_params=pltpu.CompilerParams(use_tc_tiling_on_sc=False),
)
def _lfi_gather(x_hbm, idx_hbm, out_hbm, idx_v, rows_v, sem):
    cid = lax.axis_index("c")
    sid = lax.axis_index("s")
    wid = sid * _NC + cid
    base = wid * _RPW
    off = (base // _RPB) * _N

    def chunk_body(g, carry):
        row0 = base + g * _C
        pltpu.sync_copy(idx_hbm.at[pl.ds(row0, _C)], idx_v)

        def add_body(i, c):
            s = pl.ds(i * 16, 16)
            idx_v[s] = idx_v[s] + off
            return c

        lax.fori_loop(0, _C // 16, add_body, 0)

        copies = []
        for j in range(_NG):
            copies.append(
                pltpu.async_copy(
                    x_hbm.at[idx_v.at[pl.ds(j * _G, _G)]],
                    rows_v.at[pl.ds(j * _G, _G)],
                    sem,
                )
            )
        for cp in copies:
            cp.wait()
        pltpu.sync_copy(rows_v, out_hbm.at[pl.ds(row0, _C)])
        return carry

    lax.fori_loop(0, _NCHUNK, chunk_body, 0)


def kernel(x, refer_idx):
    b, n, d = x.shape
    k = refer_idx.shape[2]
    x_flat = x.reshape(b * n, d)
    idx_flat = refer_idx.astype(jnp.int32).reshape(-1)
    out = _lfi_gather(x_flat, idx_flat)
    return out.reshape(b, n, k * d)


# speedup vs baseline: 6.8734x; 1.0533x over previous
import functools

import jax
import jax.numpy as jnp
from jax import lax
from jax.experimental import pallas as pl
from jax.experimental.pallas import tpu as pltpu
from jax.experimental.pallas import tpu_sc as plsc

_B, _N, _K, _D = 8, 4096, 16, 64
_ROWS = _B * _N * _K
_RPB = _N * _K
_NC, _NS = 2, 16
_NW = _NC * _NS
_RPW = _ROWS // _NW
_C = 512
_NCHUNK = _RPW // _C
_G = 128
_NG = _C // _G

_mesh = plsc.VectorSubcoreMesh(core_axis_name="c", subcore_axis_name="s")


@functools.partial(
    pl.kernel,
    mesh=_mesh,
    out_type=jax.ShapeDtypeStruct((_ROWS, _D), jnp.float32),
    scratch_types=[
        pltpu.VMEM((_C,), jnp.int32),
        pltpu.VMEM((_C,), jnp.int32),
        pltpu.VMEM((_C, _D), jnp.float32),
        pltpu.VMEM((_C, _D), jnp.float32),
        pltpu.SemaphoreType.DMA,
        pltpu.SemaphoreType.DMA,
        pltpu.SemaphoreType.DMA,
        pltpu.SemaphoreType.DMA,
    ],
    compiler_params=pltpu.CompilerParams(use_tc_tiling_on_sc=False),
)
def _lfi_gather(x_hbm, idx_hbm, out_hbm, idx0, idx1, rows0, rows1,
                sem_g0, sem_g1, sem_w, sem_i):
    cid = lax.axis_index("c")
    sid = lax.axis_index("s")
    wid = sid * _NC + cid
    base = wid * _RPW
    off = (base // _RPB) * _N

    idx_v = (idx0, idx1)
    rows_v = (rows0, rows1)
    sem_g = (sem_g0, sem_g1)

    def clamp_row0(g):
        gc = jnp.minimum(g, _NCHUNK - 1)
        return base + gc * _C

    def prep(s):
        def add_body(i, c):
            sl = pl.ds(i * 16, 16)
            idx_v[s][sl] = idx_v[s][sl] + off
            return c
        lax.fori_loop(0, _C // 16, add_body, 0)

    def fire_gathers(s):
        for j in range(_NG):
            pltpu.async_copy(
                x_hbm.at[idx_v[s].at[pl.ds(j * _G, _G)]],
                rows_v[s].at[pl.ds(j * _G, _G)],
                sem_g[s],
            )

    def wait_gathers(s):
        for j in range(_NG):
            pltpu.make_async_copy(
                x_hbm.at[idx_v[s].at[pl.ds(j * _G, _G)]],
                rows_v[s].at[pl.ds(j * _G, _G)],
                sem_g[s],
            ).wait()

    def fire_idx(g, s):
        pltpu.async_copy(idx_hbm.at[pl.ds(clamp_row0(g), _C)], idx_v[s], sem_i)

    def wait_idx(g, s):
        pltpu.make_async_copy(
            idx_hbm.at[pl.ds(clamp_row0(g), _C)], idx_v[s], sem_i
        ).wait()

    def fire_wb(g, s):
        pltpu.async_copy(rows_v[s], out_hbm.at[pl.ds(base + g * _C, _C)], sem_w)

    def wait_wb(g, s):
        pltpu.make_async_copy(
            rows_v[s], out_hbm.at[pl.ds(base + g * _C, _C)], sem_w
        ).wait()

    fire_idx(0, 0)
    wait_idx(0, 0)
    prep(0)
    fire_gathers(0)
    fire_idx(1, 1)

    def step(g, s):
        wait_idx(g + 1, 1 - s)
        prep(1 - s)

        @pl.when(g > 0)
        def _():
            wait_wb(g - 1, 1 - s)

        @pl.when(g + 1 < _NCHUNK)
        def _():
            fire_gathers(1 - s)

        wait_gathers(s)
        fire_idx(g + 2, s)
        fire_wb(g, s)

    def outer(o, carry):
        step(2 * o, 0)
        step(2 * o + 1, 1)
        return carry

    lax.fori_loop(0, _NCHUNK // 2, outer, 0)

    wait_wb(_NCHUNK - 1, (_NCHUNK - 1) % 2)
    wait_idx(_NCHUNK + 1, (_NCHUNK - 1) % 2)


def kernel(x, refer_idx):
    b, n, d = x.shape
    k = refer_idx.shape[2]
    x_flat = x.reshape(b * n, d)
    idx_flat = refer_idx.astype(jnp.int32).reshape(-1)
    out = _lfi_gather(x_flat, idx_flat)
    return out.reshape(b, n, k * d)


# speedup vs baseline: 14.0186x; 2.0395x over previous
import functools

import jax
import jax.numpy as jnp
from jax import lax
from jax.experimental import pallas as pl
from jax.experimental.pallas import tpu as pltpu
from jax.experimental.pallas import tpu_sc as plsc

_B, _N, _K, _D = 8, 4096, 16, 64
_ROWS = _B * _N * _K
_RPB = _N * _K
_NC, _NS = 2, 16
_NW = _NC * _NS
_RPW = _ROWS // _NW
_C = 512
_NCHUNK = _RPW // _C
_G = 128
_NG = _C // _G

_mesh = plsc.VectorSubcoreMesh(core_axis_name="c", subcore_axis_name="s")


@functools.partial(
    pl.kernel,
    mesh=_mesh,
    out_type=jax.ShapeDtypeStruct((_ROWS, _D), jnp.float32),
    scratch_types=[
        pltpu.VMEM((_K, _C // _K), jnp.int32),
        pltpu.VMEM((_K, _C // _K), jnp.int32),
        pltpu.VMEM((_C,), jnp.int32),
        pltpu.VMEM((_C,), jnp.int32),
        pltpu.VMEM((_C, _D), jnp.float32),
        pltpu.VMEM((_C, _D), jnp.float32),
        pltpu.VMEM((_C, _D), jnp.float32),
        pltpu.SemaphoreType.DMA,
        pltpu.SemaphoreType.DMA,
        pltpu.SemaphoreType.DMA,
        pltpu.SemaphoreType.DMA,
        pltpu.SemaphoreType.DMA,
        pltpu.SemaphoreType.DMA,
        pltpu.SemaphoreType.DMA,
    ],
    compiler_params=pltpu.CompilerParams(
        use_tc_tiling_on_sc=False, needs_layout_passes=False
    ),
)
def _lfi_gather(x_hbm, idx_hbm, out_hbm, raw0, raw1, idx0, idx1,
                rows0, rows1, rows2,
                sem_g0, sem_g1, sem_g2, sem_w0, sem_w1, sem_w2, sem_i):
    cid = lax.axis_index("c")
    sid = lax.axis_index("s")
    wid = sid * _NC + cid
    base = wid * _RPW
    bb = base // _RPB
    off = bb * _N
    p_base = (base % _RPB) // _K

    raw_v = (raw0, raw1)
    idx_v = (idx0, idx1)
    rows_v = (rows0, rows1, rows2)
    sem_g = (sem_g0, sem_g1, sem_g2)
    sem_w = (sem_w0, sem_w1, sem_w2)

    def p0_of(g):
        return pl.multiple_of(p_base + g * (_C // _K), _C // _K)

    def prep(s):
        iota = lax.iota(jnp.int32, 16)

        def body(t, c):
            rows = (t & 7) * 2 + (iota & 1)
            cols = (t >> 3) * 8 + (iota >> 1)
            v = plsc.load_gather(raw_v[s], [rows, cols])
            idx_v[s][pl.ds(t * 16, 16)] = v + off
            return c

        lax.fori_loop(0, _C // 16, body, 0)

    def fire_gathers(g):
        si, sr = g % 2, g % 3
        for j in range(_NG):
            pltpu.async_copy(
                x_hbm.at[idx_v[si].at[pl.ds(j * _G, _G)]],
                rows_v[sr].at[pl.ds(j * _G, _G)],
                sem_g[sr],
            )

    def wait_gathers(g):
        si, sr = g % 2, g % 3
        for j in range(_NG):
            pltpu.make_async_copy(
                x_hbm.at[idx_v[si].at[pl.ds(j * _G, _G)]],
                rows_v[sr].at[pl.ds(j * _G, _G)],
                sem_g[sr],
            ).wait()

    def fire_idx(g):
        pltpu.async_copy(
            idx_hbm.at[bb, :, pl.ds(p0_of(g), _C // _K)], raw_v[g % 2], sem_i
        )

    def wait_idx(g):
        pltpu.make_async_copy(
            idx_hbm.at[bb, :, pl.ds(p0_of(g), _C // _K)], raw_v[g % 2], sem_i
        ).wait()

    def fire_wb(g):
        sr = g % 3
        pltpu.async_copy(
            rows_v[sr], out_hbm.at[pl.ds(base + g * _C, _C)], sem_w[sr]
        )

    def wait_wb(g):
        sr = g % 3
        pltpu.make_async_copy(
            rows_v[sr], out_hbm.at[pl.ds(base + g * _C, _C)], sem_w[sr]
        ).wait()

    fire_idx(0)
    wait_idx(0)
    prep(0)
    fire_gathers(0)
    fire_idx(1)

    for g in range(_NCHUNK):
        if g + 1 < _NCHUNK:
            wait_idx(g + 1)
            if g + 2 < _NCHUNK:
                fire_idx(g + 2)
            prep((g + 1) % 2)
            if g >= 2:
                wait_wb(g - 2)
            fire_gathers(g + 1)
        wait_gathers(g)
        fire_wb(g)

    wait_wb(_NCHUNK - 3)
    wait_wb(_NCHUNK - 2)
    wait_wb(_NCHUNK - 1)


def kernel(x, refer_idx):
    b, n, d = x.shape
    k = refer_idx.shape[2]
    x_flat = x.reshape(b * n, d)
    idx_t = refer_idx.astype(jnp.int32).transpose(0, 2, 1)
    out = _lfi_gather(x_flat, idx_t)
    out5 = out.reshape(b, n // 8, k // 2, 8, 2 * d)
    return out5.transpose(0, 1, 3, 2, 4).reshape(b, n, k * d)


# speedup vs baseline: 14.2107x; 1.0137x over previous
import functools

import jax
import jax.numpy as jnp
from jax import lax
from jax.experimental import pallas as pl
from jax.experimental.pallas import tpu as pltpu
from jax.experimental.pallas import tpu_sc as plsc

_B, _N, _K, _D = 8, 4096, 16, 64
_ROWS = _B * _N * _K
_RPB = _N * _K
_NC, _NS = 2, 16
_NW = _NC * _NS
_RPW = _ROWS // _NW
_C = 256
_NCHUNK = _RPW // _C
_G = 128
_NG = _C // _G

_mesh = plsc.VectorSubcoreMesh(core_axis_name="c", subcore_axis_name="s")


@functools.partial(
    pl.kernel,
    mesh=_mesh,
    out_type=jax.ShapeDtypeStruct((_ROWS, _D), jnp.float32),
    scratch_types=[
        pltpu.VMEM((_K, _C // _K), jnp.int32),
        pltpu.VMEM((_K, _C // _K), jnp.int32),
        pltpu.VMEM((_C,), jnp.int32),
        pltpu.VMEM((_C,), jnp.int32),
        pltpu.VMEM((_C, _D), jnp.float32),
        pltpu.VMEM((_C, _D), jnp.float32),
        pltpu.SemaphoreType.DMA,
        pltpu.SemaphoreType.DMA,
        pltpu.SemaphoreType.DMA,
        pltpu.SemaphoreType.DMA,
    ],
    compiler_params=pltpu.CompilerParams(
        use_tc_tiling_on_sc=False, needs_layout_passes=False
    ),
)
def _lfi_gather(x_hbm, idx_hbm, out_hbm, raw0, raw1, idx0, idx1, rows0, rows1,
                sem_g0, sem_g1, sem_w, sem_i):
    cid = lax.axis_index("c")
    sid = lax.axis_index("s")
    wid = sid * _NC + cid
    base = wid * _RPW
    bb = base // _RPB
    off = bb * _N
    p_base = (base % _RPB) // _K

    raw_v = (raw0, raw1)
    idx_v = (idx0, idx1)
    rows_v = (rows0, rows1)
    sem_g = (sem_g0, sem_g1)

    def clamp_p0(g):
        gc = jnp.minimum(g, _NCHUNK - 1)
        return pl.multiple_of(p_base + gc * (_C // _K), _C // _K)

    def prep(s):
        iota = lax.iota(jnp.int32, 16)

        def body(t, c):
            rows = (t & 7) * 2 + (iota & 1)
            cols = (t >> 3) * 8 + (iota >> 1)
            v = plsc.load_gather(raw_v[s], [rows, cols])
            idx_v[s][pl.ds(t * 16, 16)] = v + off
            return c

        lax.fori_loop(0, _C // 16, body, 0)

    def fire_gathers(s):
        for j in range(_NG):
            pltpu.async_copy(
                x_hbm.at[idx_v[s].at[pl.ds(j * _G, _G)]],
                rows_v[s].at[pl.ds(j * _G, _G)],
                sem_g[s],
            )

    def wait_gathers(s):
        for j in range(_NG):
            pltpu.make_async_copy(
                x_hbm.at[idx_v[s].at[pl.ds(j * _G, _G)]],
                rows_v[s].at[pl.ds(j * _G, _G)],
                sem_g[s],
            ).wait()

    def fire_idx(g, s):
        pltpu.async_copy(
            idx_hbm.at[bb, :, pl.ds(clamp_p0(g), _C // _K)], raw_v[s], sem_i
        )

    def wait_idx(g, s):
        pltpu.make_async_copy(
            idx_hbm.at[bb, :, pl.ds(clamp_p0(g), _C // _K)], raw_v[s], sem_i
        ).wait()

    def fire_wb(g, s):
        pltpu.async_copy(rows_v[s], out_hbm.at[pl.ds(base + g * _C, _C)], sem_w)

    def wait_wb(g, s):
        pltpu.make_async_copy(
            rows_v[s], out_hbm.at[pl.ds(base + g * _C, _C)], sem_w
        ).wait()

    fire_idx(0, 0)
    wait_idx(0, 0)
    prep(0)
    fire_gathers(0)
    fire_idx(1, 1)

    def step(g, s):
        wait_idx(g + 1, 1 - s)
        fire_idx(g + 2, s)
        prep(1 - s)

        @pl.when(g > 0)
        def _():
            wait_wb(g - 1, 1 - s)

        @pl.when(g + 1 < _NCHUNK)
        def _():
            fire_gathers(1 - s)

        wait_gathers(s)
        fire_wb(g, s)

    def outer(o, carry):
        step(2 * o, 0)
        step(2 * o + 1, 1)
        return carry

    lax.fori_loop(0, _NCHUNK // 2, outer, 0)

    wait_wb(_NCHUNK - 1, (_NCHUNK - 1) % 2)
    wait_idx(_NCHUNK + 1, (_NCHUNK - 1) % 2)


def kernel(x, refer_idx):
    b, n, d = x.shape
    k = refer_idx.shape[2]
    x_flat = x.reshape(b * n, d)
    idx_t = refer_idx.astype(jnp.int32).transpose(0, 2, 1)
    out = _lfi_gather(x_flat, idx_t)
    out5 = out.reshape(b, n // 8, k // 2, 8, 2 * d)
    return out5.transpose(0, 1, 3, 2, 4).reshape(b, n, k * d)


# speedup vs baseline: 14.2424x; 1.0022x over previous
import functools

import jax
import jax.numpy as jnp
from jax import lax
from jax.experimental import pallas as pl
from jax.experimental.pallas import tpu as pltpu
from jax.experimental.pallas import tpu_sc as plsc

_B, _N, _K, _D = 8, 4096, 16, 64
_ROWS = _B * _N * _K
_RPB = _N * _K
_NC, _NS = 2, 16
_NW = _NC * _NS
_RPW = _ROWS // _NW
_C = 512
_NCHUNK = _RPW // _C
_G = 64
_NG = _C // _G

_mesh = plsc.VectorSubcoreMesh(core_axis_name="c", subcore_axis_name="s")


@functools.partial(
    pl.kernel,
    mesh=_mesh,
    out_type=jax.ShapeDtypeStruct((_ROWS, _D), jnp.float32),
    scratch_types=[
        pltpu.VMEM((_K, _C // _K), jnp.int32),
        pltpu.VMEM((_K, _C // _K), jnp.int32),
        pltpu.VMEM((_C,), jnp.int32),
        pltpu.VMEM((_C,), jnp.int32),
        pltpu.VMEM((_C, _D), jnp.float32),
        pltpu.VMEM((_C, _D), jnp.float32),
        pltpu.SemaphoreType.DMA,
        pltpu.SemaphoreType.DMA,
        pltpu.SemaphoreType.DMA,
        pltpu.SemaphoreType.DMA,
    ],
    compiler_params=pltpu.CompilerParams(
        use_tc_tiling_on_sc=False, needs_layout_passes=False
    ),
)
def _lfi_gather(x_hbm, idx_hbm, out_hbm, raw0, raw1, idx0, idx1, rows0, rows1,
                sem_g0, sem_g1, sem_w, sem_i):
    cid = lax.axis_index("c")
    sid = lax.axis_index("s")
    wid = sid * _NC + cid
    base = wid * _RPW
    bb = base // _RPB
    off = bb * _N
    p_base = (base % _RPB) // _K

    raw_v = (raw0, raw1)
    idx_v = (idx0, idx1)
    rows_v = (rows0, rows1)
    sem_g = (sem_g0, sem_g1)

    def clamp_p0(g):
        gc = jnp.minimum(g, _NCHUNK - 1)
        return pl.multiple_of(p_base + gc * (_C // _K), _C // _K)

    def prep(s):
        iota = lax.iota(jnp.int32, 16)

        def body(t, c):
            rows = (t & 7) * 2 + (iota & 1)
            cols = (t >> 3) * 8 + (iota >> 1)
            v = plsc.load_gather(raw_v[s], [rows, cols])
            idx_v[s][pl.ds(t * 16, 16)] = v + off
            return c

        lax.fori_loop(0, _C // 16, body, 0)

    def fire_gathers(s):
        for j in range(_NG):
            pltpu.async_copy(
                x_hbm.at[idx_v[s].at[pl.ds(j * _G, _G)]],
                rows_v[s].at[pl.ds(j * _G, _G)],
                sem_g[s],
            )

    def wait_gathers(s):
        for j in range(_NG):
            pltpu.make_async_copy(
                x_hbm.at[idx_v[s].at[pl.ds(j * _G, _G)]],
                rows_v[s].at[pl.ds(j * _G, _G)],
                sem_g[s],
            ).wait()

    def fire_idx(g, s):
        pltpu.async_copy(
            idx_hbm.at[bb, :, pl.ds(clamp_p0(g), _C // _K)], raw_v[s], sem_i
        )

    def wait_idx(g, s):
        pltpu.make_async_copy(
            idx_hbm.at[bb, :, pl.ds(clamp_p0(g), _C // _K)], raw_v[s], sem_i
        ).wait()

    def fire_wb(g, s):
        pltpu.async_copy(rows_v[s], out_hbm.at[pl.ds(base + g * _C, _C)], sem_w)

    def wait_wb(g, s):
        pltpu.make_async_copy(
            rows_v[s], out_hbm.at[pl.ds(base + g * _C, _C)], sem_w
        ).wait()

    fire_idx(0, 0)
    wait_idx(0, 0)
    prep(0)
    fire_gathers(0)
    fire_idx(1, 1)

    def step(g, s):
        wait_idx(g + 1, 1 - s)
        fire_idx(g + 2, s)
        prep(1 - s)

        @pl.when(g > 0)
        def _():
            wait_wb(g - 1, 1 - s)

        @pl.when(g + 1 < _NCHUNK)
        def _():
            fire_gathers(1 - s)

        wait_gathers(s)
        fire_wb(g, s)

    def outer(o, carry):
        step(2 * o, 0)
        step(2 * o + 1, 1)
        return carry

    lax.fori_loop(0, _NCHUNK // 2, outer, 0)

    wait_wb(_NCHUNK - 1, (_NCHUNK - 1) % 2)
    wait_idx(_NCHUNK + 1, (_NCHUNK - 1) % 2)


def kernel(x, refer_idx):
    b, n, d = x.shape
    k = refer_idx.shape[2]
    x_flat = x.reshape(b * n, d)
    idx_t = refer_idx.astype(jnp.int32).transpose(0, 2, 1)
    out = _lfi_gather(x_flat, idx_t)
    out5 = out.reshape(b, n // 8, k // 2, 8, 2 * d)
    return out5.transpose(0, 1, 3, 2, 4).reshape(b, n, k * d)
